# baseline (speedup 1.0000x reference)
import functools

import jax
import jax.numpy as jnp
from jax import lax
from jax.experimental import pallas as pl
from jax.experimental.pallas import tpu as pltpu
from jax.experimental.pallas import tpu_sc as plsc

T = 8192
D_IN = 2048
D_HID = 8192
D_OUT = 2048
K = 2048

BT = 512

NC = 2
NS = 16
NW = NC * NS
ROWS_PER_W = K // NW
CH = 16
NCH = ROWS_PER_W // CH

_MESH = plsc.VectorSubcoreMesh(core_axis_name="c", subcore_axis_name="s")


@functools.partial(
    pl.kernel,
    mesh=_MESH,
    out_type=[
        jax.ShapeDtypeStruct((K, D_IN), jnp.float32),
        jax.ShapeDtypeStruct((K,), jnp.float32),
        jax.ShapeDtypeStruct((K, D_OUT), jnp.float32),
    ],
    scratch_types=[
        pltpu.VMEM((NCH, CH), jnp.int32),
        pltpu.VMEM((ROWS_PER_W,), jnp.int32),
        pltpu.VMEM((CH, D_IN), jnp.float32),
        pltpu.VMEM((CH, D_IN), jnp.float32),
        pltpu.VMEM((CH, D_IN), jnp.float32),
        pltpu.VMEM((ROWS_PER_W,), jnp.float32),
        pltpu.SemaphoreType.DMA,
        pltpu.SemaphoreType.DMA,
        pltpu.SemaphoreType.DMA,
        pltpu.SemaphoreType.DMA,
        pltpu.SemaphoreType.DMA,
        pltpu.SemaphoreType.DMA,
        pltpu.SemaphoreType.DMA,
    ],
)
def _sc_gather(idx3_hbm, idxf_hbm, w1_hbm, b1_hbm, w2_hbm,
               w1o, b1o, w2o,
               idx2d, idx1d, buf0, buf1, buf2, bias_v,
               semg0, semg1, semg2, semw0, semw1, semw2, semb):
    wid = lax.axis_index("s") * NC + lax.axis_index("c")
    base = wid * ROWS_PER_W
    pltpu.sync_copy(idx3_hbm.at[wid], idx2d)
    pltpu.sync_copy(idxf_hbm.at[pl.ds(base, ROWS_PER_W)], idx1d)
    bias_cp = pltpu.async_copy(b1_hbm.at[idx1d], bias_v, semb)

    tables = [(w1_hbm, w1o), (w2_hbm, w2o)]
    seq = [(t, c) for t in range(2) for c in range(NCH)]
    bufs = (buf0, buf1, buf2)
    gsems = (semg0, semg1, semg2)
    wsems = (semw0, semw1, semw2)
    NBUF = 3

    def start(k):
        t, c = seq[k]
        tab, _ = tables[t]
        return pltpu.async_copy(tab.at[idx2d.at[c]], bufs[k % NBUF],
                                gsems[k % NBUF])

    def wb_start(k):
        t, c = seq[k]
        _, out = tables[t]
        return pltpu.async_copy(bufs[k % NBUF],
                                out.at[pl.ds(base + c * CH, CH)],
                                wsems[k % NBUF])

    n = len(seq)
    g, w = {}, {}
    for k in range(n):
        if k >= NBUF:
            w[k - NBUF].wait()
        g[k] = start(k)
        if k >= 1:
            g[k - 1].wait()
            w[k - 1] = wb_start(k - 1)
    g[n - 1].wait()
    w[n - 1] = wb_start(n - 1)
    for j in range(n - NBUF + 1, n):
        w[j].wait()

    bias_cp.wait()
    pltpu.sync_copy(bias_v, b1o.at[pl.ds(base, ROWS_PER_W)])


def _mlp_body(b1_ref, b2_ref, x_ref, w1_ref, w2_hbm_ref, o_ref, w2v_ref, w2_sem):
    i = pl.program_id(0)

    @pl.when(i == 0)
    def _start_w2():
        pltpu.make_async_copy(w2_hbm_ref, w2v_ref, w2_sem).start()

    a = jax.lax.dot_general(
        x_ref[...], w1_ref[...], (((1,), (1,)), ((), ())),
        preferred_element_type=jnp.float32,
        precision=jax.lax.Precision.DEFAULT)
    h = jnp.maximum(a + b1_ref[...], 0.0)

    @pl.when(i == 0)
    def _wait_w2():
        pltpu.make_async_copy(w2_hbm_ref, w2v_ref, w2_sem).wait()

    o = jax.lax.dot_general(
        h, w2v_ref[...], (((1,), (0,)), ((), ())),
        preferred_element_type=jnp.float32,
        precision=jax.lax.Precision.DEFAULT)
    o_ref[...] = o + b2_ref[...]


def _fused_mlp(x, w1_sel, b1_sel, w2_sel, fc2_b):
    grid = (T // BT,)
    return pl.pallas_call(
        _mlp_body,
        grid=grid,
        in_specs=[
            pl.BlockSpec((1, K), lambda i: (0, 0)),
            pl.BlockSpec((1, D_OUT), lambda i: (0, 0)),
            pl.BlockSpec((BT, D_IN), lambda i: (i, 0)),
            pl.BlockSpec((K, D_IN), lambda i: (0, 0)),
            pl.BlockSpec(memory_space=pl.ANY),
        ],
        out_specs=pl.BlockSpec((BT, D_OUT), lambda i: (i, 0)),
        out_shape=jax.ShapeDtypeStruct((T, D_OUT), jnp.float32),
        scratch_shapes=[
            pltpu.VMEM((K, D_OUT), jnp.float32),
            pltpu.SemaphoreType.DMA,
        ],
        compiler_params=pltpu.CompilerParams(
            dimension_semantics=("arbitrary",),
            vmem_limit_bytes=128 * 1024 * 1024,
        ),
    )(b1_sel.reshape(1, K), fc2_b.reshape(1, D_OUT), x, w1_sel, w2_sel)


def kernel(x, index_vec, fc1_w, fc1_b, fc2_w_t, fc2_b):
    idx3 = index_vec.reshape(NW, NCH, CH)
    w1_sel, b1_sel, w2_sel = _sc_gather(idx3, index_vec, fc1_w, fc1_b, fc2_w_t)
    return _fused_mlp(x, w1_sel, b1_sel, w2_sel, fc2_b)


# speedup vs baseline: 1.0063x; 1.0063x over previous
import functools

import jax
import jax.numpy as jnp
from jax import lax
from jax.experimental import pallas as pl
from jax.experimental.pallas import tpu as pltpu
from jax.experimental.pallas import tpu_sc as plsc

T = 8192
D_IN = 2048
D_HID = 8192
D_OUT = 2048
K = 2048

BT = 512

NC = 2
NS = 16
NW = NC * NS
ROWS_PER_W = K // NW
CH = 8
NCH = ROWS_PER_W // CH

_MESH = plsc.VectorSubcoreMesh(core_axis_name="c", subcore_axis_name="s")


@functools.partial(
    pl.kernel,
    mesh=_MESH,
    out_type=[
        jax.ShapeDtypeStruct((K, D_IN), jnp.float32),
        jax.ShapeDtypeStruct((K,), jnp.float32),
        jax.ShapeDtypeStruct((K, D_OUT), jnp.float32),
    ],
    scratch_types=[
        pltpu.VMEM((NCH, CH), jnp.int32),
        pltpu.VMEM((ROWS_PER_W,), jnp.int32),
        pltpu.VMEM((CH, D_IN), jnp.float32),
        pltpu.VMEM((CH, D_IN), jnp.float32),
        pltpu.VMEM((CH, D_IN), jnp.float32),
        pltpu.VMEM((CH, D_IN), jnp.float32),
        pltpu.VMEM((ROWS_PER_W,), jnp.float32),
        pltpu.SemaphoreType.DMA,
        pltpu.SemaphoreType.DMA,
        pltpu.SemaphoreType.DMA,
        pltpu.SemaphoreType.DMA,
        pltpu.SemaphoreType.DMA,
        pltpu.SemaphoreType.DMA,
        pltpu.SemaphoreType.DMA,
        pltpu.SemaphoreType.DMA,
        pltpu.SemaphoreType.DMA,
    ],
)
def _sc_gather(idx3_hbm, idxf_hbm, w1_hbm, b1_hbm, w2_hbm,
               w1o, b1o, w2o,
               idx2d, idx1d, buf0, buf1, buf2, buf3, bias_v,
               semg0, semg1, semg2, semg3, semw0, semw1, semw2, semw3, semb):
    wid = lax.axis_index("s") * NC + lax.axis_index("c")
    base = wid * ROWS_PER_W
    pltpu.sync_copy(idx3_hbm.at[wid], idx2d)

    tables = [(w1_hbm, w1o), (w2_hbm, w2o)]
    seq = [(t, c) for t in range(2) for c in range(NCH)]
    bufs = (buf0, buf1, buf2, buf3)
    gsems = (semg0, semg1, semg2, semg3)
    wsems = (semw0, semw1, semw2, semw3)
    NBUF = 4
    DLY = 2

    def start(k):
        t, c = seq[k]
        tab, _ = tables[t]
        return pltpu.async_copy(tab.at[idx2d.at[c]], bufs[k % NBUF],
                                gsems[k % NBUF])

    def wb_start(k):
        t, c = seq[k]
        _, out = tables[t]
        return pltpu.async_copy(bufs[k % NBUF],
                                out.at[pl.ds(base + c * CH, CH)],
                                wsems[k % NBUF])

    n = len(seq)
    g, w = {}, {}
    g[0] = start(0)
    g[1] = start(1)
    pltpu.sync_copy(idxf_hbm.at[pl.ds(base, ROWS_PER_W)], idx1d)
    bias_cp = pltpu.async_copy(b1_hbm.at[idx1d], bias_v, semb)
    for k in range(2, n):
        if k >= NBUF:
            w[k - NBUF].wait()
        g[k] = start(k)
        g[k - DLY].wait()
        w[k - DLY] = wb_start(k - DLY)
    for k in range(n - DLY, n):
        g[k].wait()
        w[k] = wb_start(k)
    for j in range(n - NBUF, n):
        w[j].wait()

    bias_cp.wait()
    pltpu.sync_copy(bias_v, b1o.at[pl.ds(base, ROWS_PER_W)])


def _mlp_body(b1_ref, b2_ref, x_ref, w1_ref, w2_ref, o_ref):
    a = jax.lax.dot_general(
        x_ref[...], w1_ref[...], (((1,), (1,)), ((), ())),
        preferred_element_type=jnp.float32,
        precision=jax.lax.Precision.DEFAULT)
    h = jnp.maximum(a + b1_ref[...], 0.0)
    o = jax.lax.dot_general(
        h, w2_ref[...], (((1,), (0,)), ((), ())),
        preferred_element_type=jnp.float32,
        precision=jax.lax.Precision.DEFAULT)
    o_ref[...] = o + b2_ref[...]


def _fused_mlp(x, w1_sel, b1_sel, w2_sel, fc2_b):
    grid = (T // BT,)
    return pl.pallas_call(
        _mlp_body,
        grid=grid,
        in_specs=[
            pl.BlockSpec((1, K), lambda i: (0, 0)),
            pl.BlockSpec((1, D_OUT), lambda i: (0, 0)),
            pl.BlockSpec((BT, D_IN), lambda i: (i, 0)),
            pl.BlockSpec((K, D_IN), lambda i: (0, 0)),
            pl.BlockSpec((K, D_OUT), lambda i: (0, 0)),
        ],
        out_specs=pl.BlockSpec((BT, D_OUT), lambda i: (i, 0)),
        out_shape=jax.ShapeDtypeStruct((T, D_OUT), jnp.float32),
        compiler_params=pltpu.CompilerParams(
            dimension_semantics=("arbitrary",),
            vmem_limit_bytes=128 * 1024 * 1024,
        ),
    )(b1_sel.reshape(1, K), fc2_b.reshape(1, D_OUT), x, w1_sel, w2_sel)


def kernel(x, index_vec, fc1_w, fc1_b, fc2_w_t, fc2_b):
    idx3 = index_vec.reshape(NW, NCH, CH)
    w1_sel, b1_sel, w2_sel = _sc_gather(idx3, index_vec, fc1_w, fc1_b, fc2_w_t)
    return _fused_mlp(x, w1_sel, b1_sel, w2_sel, fc2_b)
